# initial kernel scaffold (unmeasured)
import jax
import jax.numpy as jnp
from jax import lax
from jax.experimental import pallas as pl
from jax.experimental.pallas import tpu as pltpu


def kernel(
    x,
):
    def body(*refs):
        pass

    out_shape = jax.ShapeDtypeStruct(..., jnp.float32)
    return pl.pallas_call(body, out_shape=out_shape)(...)


# baseline (device time: 50790 ns/iter reference)
import jax
import jax.numpy as jnp
from jax import lax
from jax.experimental import pallas as pl
from jax.experimental.pallas import tpu as pltpu


def kernel(x):
    m, n = x.shape

    def body(x_ref, out_ref, row_halo, col_halo, col_send, send_sems, recv_sems):
        my_x = lax.axis_index("x")
        my_y = lax.axis_index("y")
        x_nbr = (1 - my_x, my_y)
        y_nbr = (my_x, 1 - my_y)

        barrier_sem = pltpu.get_barrier_semaphore()
        for nbr in (x_nbr, y_nbr):
            pl.semaphore_signal(
                barrier_sem, inc=1, device_id=nbr,
                device_id_type=pl.DeviceIdType.MESH,
            )
        pl.semaphore_wait(barrier_sem, 2)

        @pl.when(my_y == 0)
        def _():
            col_send[:, :] = x_ref[:, n - 1:n]

        @pl.when(my_y == 1)
        def _():
            col_send[:, :] = x_ref[:, 0:1]

        r_send = jnp.where(my_x == 0, m - 1, 0)

        rdma_row = pltpu.make_async_remote_copy(
            src_ref=x_ref.at[pl.ds(r_send, 1), :],
            dst_ref=row_halo,
            send_sem=send_sems.at[0],
            recv_sem=recv_sems.at[0],
            device_id=x_nbr,
            device_id_type=pl.DeviceIdType.MESH,
        )
        rdma_col = pltpu.make_async_remote_copy(
            src_ref=col_send,
            dst_ref=col_halo,
            send_sem=send_sems.at[1],
            recv_sem=recv_sems.at[1],
            device_id=y_nbr,
            device_id_type=pl.DeviceIdType.MESH,
        )
        rdma_row.start()
        rdma_col.start()
        rdma_row.wait()
        rdma_col.wait()

        xv = x_ref[:, :]
        hrow = row_halo[:, :]
        hcol = col_halo[:, :]

        up = jnp.concatenate([hrow, xv[:-1, :]], axis=0)
        down = jnp.concatenate([xv[1:, :], hrow], axis=0)
        left = jnp.concatenate([hcol, xv[:, :-1]], axis=1)
        right = jnp.concatenate([xv[:, 1:], hcol], axis=1)
        sten = 0.5 * xv + 0.125 * (up + down + left + right)

        row_ids = lax.broadcasted_iota(jnp.int32, (m, n), 0)
        col_ids = lax.broadcasted_iota(jnp.int32, (m, n), 1)
        gboundary = (
            ((my_x == 0) & (row_ids == 0))
            | ((my_x == 1) & (row_ids == m - 1))
            | ((my_y == 0) & (col_ids == 0))
            | ((my_y == 1) & (col_ids == n - 1))
        )
        out_ref[:, :] = jnp.where(gboundary, xv, sten)

    return pl.pallas_call(
        body,
        out_shape=jax.ShapeDtypeStruct((m, n), x.dtype),
        in_specs=[pl.BlockSpec(memory_space=pltpu.VMEM)],
        out_specs=pl.BlockSpec(memory_space=pltpu.VMEM),
        scratch_shapes=[
            pltpu.VMEM((1, n), x.dtype),
            pltpu.VMEM((m, 1), x.dtype),
            pltpu.VMEM((m, 1), x.dtype),
            pltpu.SemaphoreType.DMA((2,)),
            pltpu.SemaphoreType.DMA((2,)),
        ],
        compiler_params=pltpu.CompilerParams(
            collective_id=0,
            vmem_limit_bytes=100 * 1024 * 1024,
        ),
    )(x)


# device time: 40177 ns/iter; 1.2642x vs baseline; 1.2642x over previous
import jax
import jax.numpy as jnp
from jax import lax
from jax.experimental import pallas as pl
from jax.experimental.pallas import tpu as pltpu


def kernel(x):
    m, n = x.shape

    def body(x_ref, out_ref, row_halo, col_halo, col_send, send_sems, recv_sems):
        my_x = lax.axis_index("x")
        my_y = lax.axis_index("y")
        x_nbr = (1 - my_x, my_y)
        y_nbr = (my_x, 1 - my_y)

        barrier_sem = pltpu.get_barrier_semaphore()
        for nbr in (x_nbr, y_nbr):
            pl.semaphore_signal(
                barrier_sem, inc=1, device_id=nbr,
                device_id_type=pl.DeviceIdType.MESH,
            )
        pl.semaphore_wait(barrier_sem, 2)

        @pl.when(my_y == 0)
        def _():
            col_send[:, :] = x_ref[:, n - 1:n]

        @pl.when(my_y == 1)
        def _():
            col_send[:, :] = x_ref[:, 0:1]

        r_send = jnp.where(my_x == 0, m - 1, 0)

        rdma_row = pltpu.make_async_remote_copy(
            src_ref=x_ref.at[pl.ds(r_send, 1), :],
            dst_ref=row_halo,
            send_sem=send_sems.at[0],
            recv_sem=recv_sems.at[0],
            device_id=x_nbr,
            device_id_type=pl.DeviceIdType.MESH,
        )
        rdma_col = pltpu.make_async_remote_copy(
            src_ref=col_send,
            dst_ref=col_halo,
            send_sem=send_sems.at[1],
            recv_sem=recv_sems.at[1],
            device_id=y_nbr,
            device_id_type=pl.DeviceIdType.MESH,
        )
        rdma_row.start()
        rdma_col.start()

        xv = x_ref[:, :]
        out_ref[:, :] = 0.5 * xv + 0.125 * pltpu.roll(xv, 1, 0)
        out_ref[:, :] = out_ref[:, :] + 0.125 * pltpu.roll(xv, m - 1, 0)
        out_ref[:, :] = out_ref[:, :] + 0.125 * pltpu.roll(xv, 1, 1)
        out_ref[:, :] = out_ref[:, :] + 0.125 * pltpu.roll(xv, n - 1, 1)

        rdma_row.wait()
        rdma_col.wait()

        hrow = row_halo[:, :]
        hcol = col_halo[:, :]

        @pl.when(my_x == 1)
        def _():
            out_ref[0:1, :] = out_ref[0:1, :] + 0.125 * (hrow - x_ref[m - 1:m, :])

        @pl.when(my_x == 0)
        def _():
            out_ref[m - 1:m, :] = out_ref[m - 1:m, :] + 0.125 * (hrow - x_ref[0:1, :])

        @pl.when(my_y == 1)
        def _():
            out_ref[:, 0:1] = out_ref[:, 0:1] + 0.125 * (hcol - x_ref[:, n - 1:n])

        @pl.when(my_y == 0)
        def _():
            out_ref[:, n - 1:n] = out_ref[:, n - 1:n] + 0.125 * (hcol - x_ref[:, 0:1])

        @pl.when(my_x == 0)
        def _():
            out_ref[0:1, :] = x_ref[0:1, :]

        @pl.when(my_x == 1)
        def _():
            out_ref[m - 1:m, :] = x_ref[m - 1:m, :]

        @pl.when(my_y == 0)
        def _():
            out_ref[:, 0:1] = x_ref[:, 0:1]

        @pl.when(my_y == 1)
        def _():
            out_ref[:, n - 1:n] = x_ref[:, n - 1:n]

    return pl.pallas_call(
        body,
        out_shape=jax.ShapeDtypeStruct((m, n), x.dtype),
        in_specs=[pl.BlockSpec(memory_space=pltpu.VMEM)],
        out_specs=pl.BlockSpec(memory_space=pltpu.VMEM),
        scratch_shapes=[
            pltpu.VMEM((1, n), x.dtype),
            pltpu.VMEM((m, 1), x.dtype),
            pltpu.VMEM((m, 1), x.dtype),
            pltpu.SemaphoreType.DMA((2,)),
            pltpu.SemaphoreType.DMA((2,)),
        ],
        compiler_params=pltpu.CompilerParams(
            collective_id=0,
            vmem_limit_bytes=100 * 1024 * 1024,
        ),
    )(x)
